# Initial kernel scaffold; baseline (speedup 1.0000x reference)
import jax
import jax.numpy as jnp
from jax.experimental import pallas as pl


def kernel(x, edge_index, edge_weight, e1_w, e1_b, e2_w, e2_b, e3_w, e3_b, mu_w, mu_b, ls_w, ls_b, d1_w, d1_b, d2_w, d2_b, d3_w, d3_b, xb_w, xb_b, gw1, gw2, gw3, gw4, gw5, mu_c):
    raise NotImplementedError("write your pallas kernel here")


# speedup vs baseline: 3.5292x; 3.5292x over previous
import functools

import jax
import jax.numpy as jnp
from jax import lax
from jax.experimental import pallas as pl
from jax.experimental.pallas import tpu as pltpu
from jax.experimental.pallas import tpu_sc as plsc

N = 10000
E = 160000
SIGMA = 0.5
BM = 1000


def _linear_body(a_ref, w_ref, *rest, act, has_bias, has_mix):
    i = 0
    b_ref = mix_ref = None
    if has_bias:
        b_ref = rest[i]; i += 1
    if has_mix:
        mix_ref = rest[i]; i += 1
    out_ref = rest[i]
    acc = jnp.dot(a_ref[...], w_ref[...], preferred_element_type=jnp.float32)
    if has_bias:
        acc = acc + b_ref[...]
    if act == "relu":
        acc = jnp.maximum(acc, 0.0)
    elif act == "sigmoid":
        acc = jax.nn.sigmoid(acc)
    if has_mix:
        acc = (1.0 - SIGMA) * acc + SIGMA * mix_ref[...]
    out_ref[...] = acc


def _linear(a, w, b=None, act=None, mix=None):
    n, k = a.shape
    nout = w.shape[1]
    grid = (n // BM,)
    in_specs = [
        pl.BlockSpec((BM, k), lambda i: (i, 0)),
        pl.BlockSpec((k, nout), lambda i: (0, 0)),
    ]
    ops = [a, w]
    if b is not None:
        in_specs.append(pl.BlockSpec((1, nout), lambda i: (0, 0)))
        ops.append(b)
    if mix is not None:
        in_specs.append(pl.BlockSpec((BM, nout), lambda i: (i, 0)))
        ops.append(mix)
    return pl.pallas_call(
        functools.partial(_linear_body, act=act, has_bias=b is not None,
                          has_mix=mix is not None),
        grid=grid,
        in_specs=in_specs,
        out_specs=pl.BlockSpec((BM, nout), lambda i: (i, 0)),
        out_shape=jax.ShapeDtypeStruct((n, nout), jnp.float32),
    )(*ops)


def _zq_body(p_ref, eps_ref, muct_ref, z_ref, q_ref):
    mu = p_ref[:, 0:128]
    logvar = p_ref[:, 128:256]
    z = eps_ref[...] * jnp.exp(logvar * 0.5) + mu
    z_ref[...] = z
    zm = jnp.dot(z, muct_ref[...], preferred_element_type=jnp.float32)
    r = jnp.sum(z * z, axis=1, keepdims=True)
    s = jnp.sum(muct_ref[...] * muct_ref[...], axis=0, keepdims=True)
    dist = r - 2.0 * zm + s
    q = 1.0 / (1.0 + dist)
    col = lax.broadcasted_iota(jnp.int32, q.shape, 1)
    q = jnp.where(col < 10, q, 0.0)
    q_ref[...] = q / jnp.sum(q, axis=1, keepdims=True)


def _zq(p, eps, muct):
    return pl.pallas_call(
        _zq_body,
        grid=(N // BM,),
        in_specs=[
            pl.BlockSpec((BM, 256), lambda i: (i, 0)),
            pl.BlockSpec((BM, 128), lambda i: (i, 0)),
            pl.BlockSpec((128, 128), lambda i: (0, 0)),
        ],
        out_specs=[
            pl.BlockSpec((BM, 128), lambda i: (i, 0)),
            pl.BlockSpec((BM, 128), lambda i: (i, 0)),
        ],
        out_shape=[
            jax.ShapeDtypeStruct((N, 128), jnp.float32),
            jax.ShapeDtypeStruct((N, 128), jnp.float32),
        ],
    )(p, eps, muct)


def _g5_body(s4_ref, z_ref, w_ref, out_ref):
    m = (1.0 - SIGMA) * jnp.maximum(s4_ref[...], 0.0) + SIGMA * z_ref[...]
    out_ref[...] = jnp.dot(m, w_ref[...], preferred_element_type=jnp.float32)


def _g5(s4, z128, w128):
    return pl.pallas_call(
        _g5_body,
        grid=(N // BM,),
        in_specs=[
            pl.BlockSpec((BM, 128), lambda i: (i, 0)),
            pl.BlockSpec((BM, 128), lambda i: (i, 0)),
            pl.BlockSpec((128, 128), lambda i: (0, 0)),
        ],
        out_specs=pl.BlockSpec((BM, 128), lambda i: (i, 0)),
        out_shape=jax.ShapeDtypeStruct((N, 128), jnp.float32),
    )(s4, z128, w128)


def _softmax_body(x_ref, out_ref):
    x = x_ref[...]
    col = lax.broadcasted_iota(jnp.int32, x.shape, 1)
    x = jnp.where(col < 10, x, -1e30)
    m = jnp.max(x, axis=1, keepdims=True)
    e = jnp.exp(x - m)
    out_ref[...] = e / jnp.sum(e, axis=1, keepdims=True)


def _softmax10(x):
    return pl.pallas_call(
        _softmax_body,
        grid=(N // BM,),
        in_specs=[pl.BlockSpec((BM, 128), lambda i: (i, 0))],
        out_specs=pl.BlockSpec((BM, 128), lambda i: (i, 0)),
        out_shape=jax.ShapeDtypeStruct((N, 128), jnp.float32),
    )(x)


EB = 80
NP = 10240
RPT = NP // 16
RZ = 128


@functools.lru_cache(maxsize=None)
def _make_spmm(nch, fc):
    mesh = plsc.VectorSubcoreMesh(core_axis_name="c", subcore_axis_name="s")
    ept = E // 16
    nb = ept // EB

    out_type = [jax.ShapeDtypeStruct((NP, fc), jnp.float32) for _ in range(nch)]
    scratch_types = [
        pltpu.VMEM((EB,), jnp.int32),
        pltpu.VMEM((EB,), jnp.int32),
        pltpu.VMEM((EB,), jnp.float32),
        pltpu.VMEM((EB, fc), jnp.float32),
        pltpu.VMEM((RZ, fc), jnp.float32),
        pltpu.VMEM_SHARED((NP, fc), jnp.float32),
        pltpu.SemaphoreType.DMA,
    ]

    @functools.partial(pl.kernel, mesh=mesh, out_type=out_type,
                       scratch_types=scratch_types)
    def spmm_kernel(*refs):
        h_refs = refs[:nch]
        src_hbm, dst_hbm, w_hbm = refs[nch:nch + 3]
        out_refs = refs[nch + 3:nch + 3 + nch]
        src_v, dst_v, w_v, rows_v, zero_v, acc, sem = refs[nch + 3 + nch:]
        c = lax.axis_index("c")
        s = lax.axis_index("s")

        zv = jnp.zeros((16,), jnp.float32)

        def zfill(r, _):
            for j in range(fc // 16):
                zero_v[r, pl.ds(j * 16, 16)] = zv
            return 0

        lax.fori_loop(0, RZ, zfill, 0)

        for ci in range(nch):
            @pl.when(c == ci % 2)
            def _process(ci=ci):
                def zrow(k, _):
                    pltpu.sync_copy(zero_v,
                                    acc.at[pl.ds(s * RPT + k * RZ, RZ)])
                    return 0

                lax.fori_loop(0, RPT // RZ, zrow, 0)
                plsc.subcore_barrier()

                def ebatch(k, _):
                    base = s * ept + k * EB
                    pltpu.sync_copy(src_hbm.at[pl.ds(base, EB)], src_v)
                    pltpu.sync_copy(dst_hbm.at[pl.ds(base, EB)], dst_v)
                    pltpu.sync_copy(w_hbm.at[pl.ds(base, EB)], w_v)
                    pltpu.async_copy(h_refs[ci].at[src_v], rows_v, sem).wait()

                    def scale(g, _):
                        wvec = w_v[pl.ds(g * 16, 16)]
                        for t in range(16):
                            wv = wvec[t]
                            e = g * 16 + t
                            for j in range(fc // 16):
                                rows_v[e, pl.ds(j * 16, 16)] = (
                                    rows_v[e, pl.ds(j * 16, 16)] * wv)
                        return 0

                    lax.fori_loop(0, EB // 16, scale, 0)
                    pltpu.sync_copy(rows_v, acc.at[dst_v], add=True)
                    return 0

                lax.fori_loop(0, nb, ebatch, 0)
                plsc.subcore_barrier()

                pltpu.sync_copy(acc.at[pl.ds(s * RPT, RPT)],
                                out_refs[ci].at[pl.ds(s * RPT, RPT)])

    return spmm_kernel


def _spmm(h, src, dst, w):
    f = h.shape[1]
    fc = 128
    nch = f // 128
    chunks = [h] if nch == 1 else [h[:, i * 128:(i + 1) * 128]
                                   for i in range(nch)]
    outs = _make_spmm(nch, fc)(*chunks, src, dst, w)
    outs = [o[:N] for o in outs]
    if nch == 1:
        return outs[0]
    return jnp.concatenate(outs, axis=1)


def _padw(w, ki, ko):
    return jnp.pad(w, ((0, ki - w.shape[0]), (0, ko - w.shape[1])))


def _padb(b, ko):
    return jnp.pad(b, (0, ko - b.shape[0]))[None, :]


def kernel(x, edge_index, edge_weight, e1_w, e1_b, e2_w, e2_b, e3_w, e3_b,
           mu_w, mu_b, ls_w, ls_b, d1_w, d1_b, d2_w, d2_b, d3_w, d3_b,
           xb_w, xb_b, gw1, gw2, gw3, gw4, gw5, mu_c):
    x = x.astype(jnp.float32)
    src = edge_index[1].astype(jnp.int32)
    dst = edge_index[0].astype(jnp.int32)
    w = edge_weight.astype(jnp.float32)

    tra1 = _linear(x, _padw(e1_w, 256, 512), _padb(e1_b, 512), act="relu")
    tra2 = _linear(tra1, _padw(e2_w, 512, 512), _padb(e2_b, 512), act="relu")
    tra3 = _linear(tra2, _padw(e3_w, 512, 2048), _padb(e3_b, 2048), act="relu")

    pw = jnp.zeros((2048, 256), jnp.float32)
    pw = pw.at[:2000, 0:10].set(mu_w).at[:2000, 128:138].set(ls_w)
    pb = jnp.zeros((256,), jnp.float32)
    pb = pb.at[0:10].set(mu_b).at[128:138].set(ls_b)
    p = _linear(tra3, pw, pb[None, :], act=None)

    eps = jax.random.normal(jax.random.key(42), (N, 10), dtype=jnp.float32)
    eps128 = jnp.pad(eps, ((0, 0), (0, 118)))
    muct = jnp.zeros((128, 128), jnp.float32).at[:10, :10].set(mu_c.T)
    z128, q128 = _zq(p, eps128, muct)

    h1 = _linear(z128, _padw(d1_w, 128, 2048), _padb(d1_b, 2048), act="relu")
    h2 = _linear(h1, _padw(d2_w, 2048, 512), _padb(d2_b, 512), act="relu")
    h3 = _linear(h2, _padw(d3_w, 512, 512), _padb(d3_b, 512), act="relu")
    x_bar = _linear(h3, _padw(xb_w, 512, 256), _padb(xb_b, 256), act="sigmoid")

    sx = _spmm(x, src, dst, w)
    m2 = _linear(sx, _padw(gw1, 256, 512), act="relu", mix=tra1)
    s2 = _spmm(m2, src, dst, w)
    m3 = _linear(s2, _padw(gw2, 512, 512), act="relu", mix=tra2)
    s3 = _spmm(m3, src, dst, w)
    m4 = _linear(s3, _padw(gw3, 512, 2048), act="relu", mix=tra3)
    c4 = _linear(m4, _padw(gw4, 2048, 128), act=None)
    s4 = _spmm(c4, src, dst, w)
    c5 = _g5(s4, z128, _padw(gw5, 128, 128))
    s5 = _spmm(c5, src, dst, w)
    predict = _softmax10(s5)[:, :10]

    mu = p[:, 0:10]
    logvar = p[:, 128:138]
    return (mu, logvar, z128[:, :10], x_bar, predict, q128[:, :10])
